# baseline (speedup 1.0000x reference)
import functools

import jax
import jax.numpy as jnp
from jax import lax
from jax.experimental import pallas as pl
from jax.experimental.pallas import tpu as pltpu
from jax.experimental.pallas import tpu_sc as plsc

_G = 128
_HALF = 400
_NBUF = 4


def _make_gather(N, V, D):
    info = plsc.get_sparse_core_info()
    NC, NS = info.num_cores, info.num_subcores
    NW = NC * NS
    n_per_w = N // NW
    rows_per_w = n_per_w // _G
    n_chunks = n_per_w // _HALF
    n_outer = (n_chunks + _NBUF - 1) // _NBUF
    mesh = plsc.VectorSubcoreMesh(core_axis_name="c", subcore_axis_name="s")

    @functools.partial(
        pl.kernel,
        mesh=mesh,
        out_type=jax.ShapeDtypeStruct((N, D), jnp.float32),
        compiler_params=pltpu.CompilerParams(use_tc_tiling_on_sc=False),
        scratch_types=[
            pltpu.VMEM((n_per_w,), jnp.int32),
            pltpu.VMEM((_NBUF, _HALF, D), jnp.float32),
            pltpu.SemaphoreType.DMA,
            pltpu.SemaphoreType.DMA,
            pltpu.SemaphoreType.DMA,
            pltpu.SemaphoreType.DMA,
            pltpu.SemaphoreType.DMA,
        ],
    )
    def k(idx_hbm, table_hbm, out_hbm, idx_v, rows_v, gsem, s0, s1, s2, s3):
        wid = lax.axis_index("s") * NC + lax.axis_index("c")
        base = wid * n_per_w
        ssems = (s0, s1, s2, s3)

        pltpu.sync_copy(
            idx_hbm.at[pl.ds(pl.multiple_of(base, 8), n_per_w)], idx_v)

        def fire_gather(t, b):
            pltpu.async_copy(
                table_hbm.at[idx_v.at[pl.ds(t * _HALF, _HALF)]],
                rows_v.at[b],
                gsem)

        def drain_gather(b):
            pltpu.make_async_copy(
                table_hbm.at[pl.ds(0, _HALF)], rows_v.at[b], gsem).wait()

        def fire_store(t, b):
            off = pl.multiple_of(base + t * _HALF, _HALF)
            pltpu.async_copy(rows_v.at[b], out_hbm.at[pl.ds(off, _HALF)],
                             ssems[b])

        def drain_store(b):
            pltpu.make_async_copy(
                table_hbm.at[pl.ds(0, _HALF)], rows_v.at[b], ssems[b]).wait()

        fire_gather(0, 0)
        fire_gather(1, 1)
        fire_gather(2, 2)

        def body(i, _):
            g = i * _NBUF
            for u in range(_NBUF):
                t = g + u
                b = u
                b_next = (u + 3) % _NBUF

                @pl.when(t < n_chunks)
                def _():
                    drain_gather(b)
                    fire_store(t, b)

                @pl.when(jnp.logical_and(t >= 1, t + 3 < n_chunks))
                def _():
                    drain_store(b_next)

                @pl.when(t + 3 < n_chunks)
                def _():
                    fire_gather(t + 3, b_next)
            return ()

        lax.fori_loop(0, n_outer, body, ())

        for t in range(n_chunks - _NBUF, n_chunks):
            drain_store(t % _NBUF)

    return k


def kernel(token_ids, embed_matrix):
    B, H = token_ids.shape
    V, D = embed_matrix.shape
    N = B * H
    idx_flat = token_ids.astype(jnp.int32).reshape(N)
    out = _make_gather(N, V, D)(idx_flat, embed_matrix)
    return out.reshape(B, H, D)


# speedup vs baseline: 1.0015x; 1.0015x over previous
import functools

import jax
import jax.numpy as jnp
from jax import lax
from jax.experimental import pallas as pl
from jax.experimental.pallas import tpu as pltpu
from jax.experimental.pallas import tpu_sc as plsc

_G = 128
_HALF = 400
_NBUF = 4


def _make_gather(N, V, D):
    info = plsc.get_sparse_core_info()
    NC, NS = info.num_cores, info.num_subcores
    NW = NC * NS
    n_per_w = N // NW
    rows_per_w = n_per_w // _G
    n_chunks = n_per_w // _HALF
    n_outer = (n_chunks + _NBUF - 1) // _NBUF
    mesh = plsc.VectorSubcoreMesh(core_axis_name="c", subcore_axis_name="s")

    @functools.partial(
        pl.kernel,
        mesh=mesh,
        out_type=jax.ShapeDtypeStruct((N, D), jnp.float32),
        compiler_params=pltpu.CompilerParams(use_tc_tiling_on_sc=False),
        scratch_types=[
            pltpu.VMEM((n_per_w,), jnp.int32),
            pltpu.VMEM((_NBUF, _HALF, D), jnp.float32),
            pltpu.SemaphoreType.DMA,
            pltpu.SemaphoreType.DMA,
            pltpu.SemaphoreType.DMA,
            pltpu.SemaphoreType.DMA,
            pltpu.SemaphoreType.DMA,
        ],
    )
    def k(idx_hbm, table_hbm, out_hbm, idx_v, rows_v, gsem, s0, s1, s2, s3):
        wid = lax.axis_index("s") * NC + lax.axis_index("c")
        base = wid * n_per_w
        ssems = (s0, s1, s2, s3)

        head = 3 * _HALF
        pltpu.sync_copy(
            idx_hbm.at[pl.ds(pl.multiple_of(base, 8), head)],
            idx_v.at[pl.ds(0, head)])

        def fire_gather(t, b):
            pltpu.async_copy(
                table_hbm.at[idx_v.at[pl.ds(t * _HALF, _HALF)]],
                rows_v.at[b],
                gsem)

        def drain_gather(b):
            pltpu.make_async_copy(
                table_hbm.at[pl.ds(0, _HALF)], rows_v.at[b], gsem).wait()

        def fire_store(t, b):
            off = pl.multiple_of(base + t * _HALF, _HALF)
            pltpu.async_copy(rows_v.at[b], out_hbm.at[pl.ds(off, _HALF)],
                             ssems[b])

        def drain_store(b):
            pltpu.make_async_copy(
                table_hbm.at[pl.ds(0, _HALF)], rows_v.at[b], ssems[b]).wait()

        fire_gather(0, 0)
        fire_gather(1, 1)
        fire_gather(2, 2)
        pltpu.async_copy(
            idx_hbm.at[pl.ds(pl.multiple_of(base + head, 8), n_per_w - head)],
            idx_v.at[pl.ds(head, n_per_w - head)], s3).wait()

        def body(i, _):
            g = i * _NBUF
            for u in range(_NBUF):
                t = g + u
                b = u
                b_next = (u + 3) % _NBUF

                @pl.when(t < n_chunks)
                def _():
                    drain_gather(b)
                    fire_store(t, b)

                @pl.when(jnp.logical_and(t >= 1, t + 3 < n_chunks))
                def _():
                    drain_store(b_next)

                @pl.when(t + 3 < n_chunks)
                def _():
                    fire_gather(t + 3, b_next)
            return ()

        lax.fori_loop(0, n_outer, body, ())

        for t in range(n_chunks - _NBUF, n_chunks):
            drain_store(t % _NBUF)

    return k


def kernel(token_ids, embed_matrix):
    B, H = token_ids.shape
    V, D = embed_matrix.shape
    N = B * H
    idx_flat = token_ids.astype(jnp.int32).reshape(N)
    out = _make_gather(N, V, D)(idx_flat, embed_matrix)
    return out.reshape(B, H, D)


# speedup vs baseline: 1.0017x; 1.0002x over previous
import functools

import jax
import jax.numpy as jnp
from jax import lax
from jax.experimental import pallas as pl
from jax.experimental.pallas import tpu as pltpu
from jax.experimental.pallas import tpu_sc as plsc

_HALF = 400
_NBUF = 4


def _make_gather(N, V, D):
    info = plsc.get_sparse_core_info()
    NC, NS = info.num_cores, info.num_subcores
    NW = NC * NS
    n_per_w = N // NW
    n_chunks = n_per_w // _HALF
    n_outer = (n_chunks + _NBUF - 1) // _NBUF
    mesh = plsc.VectorSubcoreMesh(core_axis_name="c", subcore_axis_name="s")

    @functools.partial(
        pl.kernel,
        mesh=mesh,
        out_type=jax.ShapeDtypeStruct((N, D), jnp.float32),
        compiler_params=pltpu.CompilerParams(use_tc_tiling_on_sc=False),
        scratch_types=[
            pltpu.VMEM((n_per_w,), jnp.int32),
            pltpu.VMEM((_NBUF, _HALF, D), jnp.float32),
            pltpu.SemaphoreType.DMA,
            pltpu.SemaphoreType.DMA,
            pltpu.SemaphoreType.DMA,
            pltpu.SemaphoreType.DMA,
            pltpu.SemaphoreType.DMA,
        ],
    )
    def k(idx_hbm, table_hbm, out_hbm, idx_v, rows_v, gsem, s0, s1, s2, s3):
        wid = lax.axis_index("s") * NC + lax.axis_index("c")
        base = wid * n_per_w
        ssems = (s0, s1, s2, s3)

        head = 3 * _HALF
        pltpu.sync_copy(
            idx_hbm.at[pl.ds(pl.multiple_of(base, 8), head)],
            idx_v.at[pl.ds(0, head)])

        def fire_gather(t, b):
            pltpu.async_copy(
                table_hbm.at[idx_v.at[pl.ds(t * _HALF, _HALF)]],
                rows_v.at[b],
                gsem)

        def drain_gather(b):
            pltpu.make_async_copy(
                table_hbm.at[pl.ds(0, _HALF)], rows_v.at[b], gsem).wait()

        def fire_store(t, b):
            off = pl.multiple_of(base + t * _HALF, _HALF)
            pltpu.async_copy(rows_v.at[b], out_hbm.at[pl.ds(off, _HALF)],
                             ssems[b])

        def drain_store(b):
            pltpu.make_async_copy(
                table_hbm.at[pl.ds(0, _HALF)], rows_v.at[b], ssems[b]).wait()

        fire_gather(0, 0)
        fire_gather(1, 1)
        fire_gather(2, 2)
        pltpu.async_copy(
            idx_hbm.at[pl.ds(pl.multiple_of(base + head, 8), n_per_w - head)],
            idx_v.at[pl.ds(head, n_per_w - head)], s3).wait()

        def body(i, _):
            g = i * _NBUF
            for u in range(_NBUF):
                t = g + u
                b = u
                b_next = (u + 3) % _NBUF

                @pl.when(t < n_chunks)
                def _():
                    drain_gather(b)
                    fire_store(t, b)

                @pl.when(jnp.logical_and(t >= 1, t + 3 < n_chunks))
                def _():
                    drain_store(b_next)

                @pl.when(t + 3 < n_chunks)
                def _():
                    fire_gather(t + 3, b_next)
            return ()

        lax.fori_loop(0, n_outer, body, ())

        for t in range(n_chunks - _NBUF, n_chunks):
            drain_store(t % _NBUF)

    return k


def kernel(token_ids, embed_matrix):
    B, H = token_ids.shape
    V, D = embed_matrix.shape
    N = B * H
    idx_flat = token_ids.astype(jnp.int32).reshape(N)
    out = _make_gather(N, V, D)(idx_flat, embed_matrix)
    return out.reshape(B, H, D)
